# Initial kernel scaffold; baseline (speedup 1.0000x reference)
import jax
import jax.numpy as jnp
from jax.experimental import pallas as pl


def kernel(x, edge_index, batch, W1, b1, W2, b2, W3, b3, Wl, bl):
    raise NotImplementedError("write your pallas kernel here")


# speedup vs baseline: 8.2139x; 8.2139x over previous
import functools

import jax
import jax.numpy as jnp
from jax import lax
from jax.experimental import pallas as pl
from jax.experimental.pallas import tpu as pltpu
from jax.experimental.pallas import tpu_sc as plsc

N = 10000
D = 128
G = 64
NC, NS = 2, 16
NW = NC * NS
CHUNK = 128
BLK = 256
N_PAD = 10240
RPT = N_PAD // NS


def _sc_mesh():
    return plsc.VectorSubcoreMesh(core_axis_name="c", subcore_axis_name="s",
                                  num_cores=NC, num_subcores=NS)


def _degree_body(nchunks, dst_hbm, zero_hbm, ones_hbm, out_hbm,
                 didx, ones_v, cnt, sem):
    cid = lax.axis_index("c")
    sid = lax.axis_index("s")
    wid = sid * NC + cid
    r0 = sid * RPT
    pltpu.sync_copy(zero_hbm.at[pl.ds(r0, RPT)], cnt.at[pl.ds(r0, RPT)])
    pltpu.sync_copy(ones_hbm, ones_v)
    plsc.subcore_barrier()

    def body(c, carry):
        base = (wid * nchunks + c) * CHUNK
        pltpu.sync_copy(dst_hbm.at[pl.ds(base, CHUNK)], didx)
        pltpu.sync_copy(ones_v, cnt.at[didx], add=True)
        return carry

    lax.fori_loop(0, nchunks, body, 0)
    plsc.subcore_barrier()
    pltpu.sync_copy(cnt.at[pl.ds(r0, RPT)],
                    out_hbm.at[pl.ds(cid * N_PAD + r0, RPT)])


def _spmm_body(nchunks, g_hbm, src_hbm, dst_hbm, zero_hbm, out_hbm,
               sidx, didx, rows, acc, sem):
    cid = lax.axis_index("c")
    sid = lax.axis_index("s")
    wid = sid * NC + cid
    r0 = sid * RPT
    pltpu.sync_copy(zero_hbm.at[pl.ds(r0, RPT)], acc.at[pl.ds(r0, RPT)])
    plsc.subcore_barrier()

    def body(c, carry):
        base = (wid * nchunks + c) * CHUNK
        pltpu.sync_copy(src_hbm.at[pl.ds(base, CHUNK)], sidx)
        pltpu.sync_copy(dst_hbm.at[pl.ds(base, CHUNK)], didx)
        pltpu.async_copy(g_hbm.at[sidx], rows, sem).wait()
        pltpu.sync_copy(rows, acc.at[didx], add=True)
        return carry

    lax.fori_loop(0, nchunks, body, 0)
    plsc.subcore_barrier()
    pltpu.sync_copy(acc.at[pl.ds(r0, RPT)],
                    out_hbm.at[pl.ds(cid * N_PAD + r0, RPT)])


def _make_degree_call(nchunks):
    return pl.kernel(
        functools.partial(_degree_body, nchunks),
        out_type=jax.ShapeDtypeStruct((2 * N_PAD, D), jnp.float32),
        mesh=_sc_mesh(),
        scratch_types=[
            pltpu.VMEM((CHUNK,), jnp.int32),
            pltpu.VMEM((CHUNK, D), jnp.float32),
            pltpu.VMEM_SHARED((N_PAD, D), jnp.float32),
            pltpu.SemaphoreType.DMA,
        ],
    )


def _make_spmm_call(nchunks):
    return pl.kernel(
        functools.partial(_spmm_body, nchunks),
        out_type=jax.ShapeDtypeStruct((2 * N_PAD, D), jnp.float32),
        mesh=_sc_mesh(),
        scratch_types=[
            pltpu.VMEM((CHUNK,), jnp.int32),
            pltpu.VMEM((CHUNK,), jnp.int32),
            pltpu.VMEM((CHUNK, D), jnp.float32),
            pltpu.VMEM_SHARED((N_PAD, D), jnp.float32),
            pltpu.SemaphoreType.DMA,
        ],
    )


def _prep_kernel(x_ref, w_ref, c0_ref, c1_ref, dinv_ref, g_ref):
    deg = 1.0 + c0_ref[:, 0:1] + c1_ref[:, 0:1]
    dinvb = jnp.broadcast_to(lax.rsqrt(deg), (BLK, D))
    dinv_ref[...] = dinvb
    g_ref[...] = jnp.dot(x_ref[...], w_ref[...],
                         preferred_element_type=jnp.float32) * dinvb


def _mid_kernel(p0_ref, p1_ref, g_ref, dinv_ref, b_ref, w_ref, gout_ref):
    dinvb = dinv_ref[...]
    s = (p0_ref[...] + p1_ref[...] + g_ref[...]) * dinvb + b_ref[...]
    h = jnp.maximum(s, 0.0)
    gout_ref[...] = jnp.dot(h, w_ref[...],
                            preferred_element_type=jnp.float32) * dinvb


def _final_kernel(p0_ref, p1_ref, g_ref, dinv_ref, b_ref, batch_ref,
                  wl_ref, bl_ref, out_ref, s_acc, c_acc):
    i = pl.program_id(0)
    nblk = pl.num_programs(0)

    @pl.when(i == 0)
    def _init():
        s_acc[...] = jnp.zeros_like(s_acc)
        c_acc[...] = jnp.zeros_like(c_acc)

    h3 = (p0_ref[...] + p1_ref[...] + g_ref[...]) * dinv_ref[...] + b_ref[...]
    onehot = (batch_ref[...] == lax.broadcasted_iota(jnp.int32, (1, G), 1)
              ).astype(jnp.float32)
    dn = (((0,), (0,)), ((), ()))
    s_acc[...] += lax.dot_general(onehot, h3, dn,
                                  preferred_element_type=jnp.float32)
    c_acc[...] += lax.dot_general(onehot, jnp.ones((BLK, D), jnp.float32), dn,
                                  preferred_element_type=jnp.float32)

    @pl.when(i == nblk - 1)
    def _fin():
        hG = s_acc[...] / jnp.maximum(c_acc[...], 1.0)
        logits = jnp.dot(hG, wl_ref[...],
                         preferred_element_type=jnp.float32) + bl_ref[...]
        m = jnp.max(logits, axis=1, keepdims=True)
        e = jnp.exp(logits - m)
        out_ref[...] = e / jnp.sum(e, axis=1, keepdims=True)


_NBLK = N_PAD // BLK

_row_spec = pl.BlockSpec((BLK, D), lambda i: (i, 0))
_full_spec = pl.BlockSpec((D, D), lambda i: (0, 0))
_vec_spec = pl.BlockSpec((1, D), lambda i: (0, 0))


def _prep_call(x_p, W1, c0, c1):
    return pl.pallas_call(
        _prep_kernel,
        grid=(_NBLK,),
        in_specs=[_row_spec, _full_spec, _row_spec, _row_spec],
        out_specs=[_row_spec, _row_spec],
        out_shape=[jax.ShapeDtypeStruct((N_PAD, D), jnp.float32),
                   jax.ShapeDtypeStruct((N_PAD, D), jnp.float32)],
    )(x_p, W1, c0, c1)


def _mid_call(p0, p1, g, dinvb, b, W):
    return pl.pallas_call(
        _mid_kernel,
        grid=(_NBLK,),
        in_specs=[_row_spec, _row_spec, _row_spec, _row_spec,
                  _vec_spec, _full_spec],
        out_specs=_row_spec,
        out_shape=jax.ShapeDtypeStruct((N_PAD, D), jnp.float32),
    )(p0, p1, g, dinvb, b, W)


def _final_call(p0, p1, g, dinvb, b, batch_p, Wl, bl):
    return pl.pallas_call(
        _final_kernel,
        grid=(_NBLK,),
        in_specs=[_row_spec, _row_spec, _row_spec, _row_spec, _vec_spec,
                  pl.BlockSpec((BLK, 1), lambda i: (i, 0)),
                  _full_spec, _vec_spec],
        out_specs=pl.BlockSpec((G, D), lambda i: (0, 0)),
        out_shape=jax.ShapeDtypeStruct((G, D), jnp.float32),
        scratch_shapes=[pltpu.VMEM((G, D), jnp.float32),
                        pltpu.VMEM((G, D), jnp.float32)],
    )(p0, p1, g, dinvb, b, batch_p, Wl, bl)


def kernel(x, edge_index, batch, W1, b1, W2, b2, W3, b3, Wl, bl):
    src = edge_index[0]
    dst = edge_index[1]
    e = src.shape[0]
    epw = CHUNK * NW
    e_pad = ((e + epw - 1) // epw) * epw
    nchunks = e_pad // epw
    pad = e_pad - e
    src_p = jnp.concatenate([src, jnp.zeros((pad,), jnp.int32)])
    dst_p = jnp.concatenate([dst, jnp.full((pad,), N, jnp.int32)])
    x_p = jnp.pad(x, ((0, N_PAD - N), (0, 0)))
    batch_p = jnp.pad(batch, (0, N_PAD - N), constant_values=G)[:, None]
    zeros_d = jnp.zeros((N_PAD, D), jnp.float32)
    ones_d = jnp.ones((CHUNK, D), jnp.float32)

    cnt = _make_degree_call(nchunks)(dst_p, zeros_d, ones_d)
    c0, c1 = cnt[:N_PAD], cnt[N_PAD:]

    dinvb, g1 = _prep_call(x_p, W1, c0, c1)

    spmm = _make_spmm_call(nchunks)
    p = spmm(g1, src_p, dst_p, zeros_d)
    g2 = _mid_call(p[:N_PAD], p[N_PAD:], g1, dinvb, b1[None, :], W2)
    p = spmm(g2, src_p, dst_p, zeros_d)
    g3 = _mid_call(p[:N_PAD], p[N_PAD:], g2, dinvb, b2[None, :], W3)
    p = spmm(g3, src_p, dst_p, zeros_d)
    return _final_call(p[:N_PAD], p[N_PAD:], g3, dinvb, b3[None, :],
                       batch_p, Wl, bl[None, :])
